# baseline (speedup 1.0000x reference)
import functools

import jax
import jax.numpy as jnp
from jax import lax
from jax.experimental import pallas as pl
from jax.experimental.pallas import tpu as pltpu
from jax.experimental.pallas import tpu_sc as plsc

B = 16384
D = 128
NC = 2
NS = 16
NW = NC * NS
PER_W = B // NW
CHUNK = 128
CHUNKS = PER_W // CHUNK


SEG = 256
SEG_CHUNKS = SEG // CHUNK
NSEG = 2 * PER_W // SEG


def _gather_body(n, uidx_hbm, iidx_hbm, utab_hbm, itab_hbm, out_u, out_i,
                 uidx_v, iidx_v, rows_a, rows_b, gsem_a, gsem_b,
                 osem_a, osem_b):
    per_w = n // NW
    wid = lax.axis_index("s") * NC + lax.axis_index("c")
    base = wid * per_w
    hu = pltpu.async_copy(uidx_hbm.at[pl.ds(base, per_w)],
                          uidx_v.at[pl.ds(0, per_w)], gsem_a)
    hi = pltpu.async_copy(iidx_hbm.at[pl.ds(base, per_w)],
                          iidx_v.at[pl.ds(0, per_w)], gsem_b)
    hu.wait()
    hi.wait()
    bufs = [rows_a, rows_b]
    gsems = [gsem_a, gsem_b]
    osems = [osem_a, osem_b]
    segs = []
    for idxv, tab, outref in ((uidx_v, utab_hbm, out_u),
                              (iidx_v, itab_hbm, out_i)):
        for s in range(per_w // SEG):
            segs.append((idxv, tab, s * SEG_CHUNKS, outref, base + s * SEG))
    nseg = len(segs)
    gh = [None] * nseg
    oh = [None] * nseg
    for k in range(nseg + 1):
        if k < nseg:
            if k >= 2:
                oh[k - 2].wait()
            idxv, tab, coff, _, _ = segs[k]
            b = k % 2
            gh[k] = [pltpu.async_copy(
                tab.at[idxv.at[pl.ds((coff + j) * CHUNK, CHUNK)]],
                bufs[b].at[pl.ds(j * CHUNK, CHUNK)], gsems[b])
                for j in range(SEG_CHUNKS)]
        if k >= 1:
            p = k - 1
            for h in gh[p]:
                h.wait()
            _, _, _, outref, roff = segs[p]
            oh[p] = pltpu.async_copy(bufs[p % 2], outref.at[pl.ds(roff, SEG)],
                                     osems[p % 2])
    if nseg >= 2:
        oh[nseg - 2].wait()
    oh[nseg - 1].wait()


def _gather(uidx, iidx, utab, itab):
    n = uidx.shape[0]
    mesh = plsc.VectorSubcoreMesh(core_axis_name="c", subcore_axis_name="s")
    k = functools.partial(
        pl.kernel,
        mesh=mesh,
        out_type=[jax.ShapeDtypeStruct((n, D), jnp.float32),
                  jax.ShapeDtypeStruct((n, D), jnp.float32)],
        scratch_types=[
            pltpu.VMEM((PER_W,), jnp.int32),
            pltpu.VMEM((PER_W,), jnp.int32),
            pltpu.VMEM((SEG, D), jnp.float32),
            pltpu.VMEM((SEG, D), jnp.float32),
            pltpu.SemaphoreType.DMA,
            pltpu.SemaphoreType.DMA,
            pltpu.SemaphoreType.DMA,
            pltpu.SemaphoreType.DMA,
        ],
    )(functools.partial(_gather_body, n))
    return k(uidx, iidx, utab, itab)


def _mlp_body(eu, ei, w1a, w1b, b1, w2, b2, wp, bp, out):
    x = (jnp.dot(eu[...], w1a[...], preferred_element_type=jnp.float32)
         + jnp.dot(ei[...], w1b[...], preferred_element_type=jnp.float32)
         + b1[...])
    h = jnp.maximum(x, 0.0)
    h2 = jnp.maximum(
        jnp.dot(h, w2[...], preferred_element_type=jnp.float32) + b2[...], 0.0)
    out[...] = (jnp.dot(wp[...], h2.T, preferred_element_type=jnp.float32)
                + bp[0, 0])[None]


def _mlp(eu, ei, w1a, w1b, b1, w2, b2, wp, bp):
    n = eu.shape[0]
    BLK = 4096
    grid = (n // BLK,)
    full = lambda i: (0, 0)
    return pl.pallas_call(
        _mlp_body,
        grid=grid,
        in_specs=[
            pl.BlockSpec((BLK, D), lambda i: (i, 0)),
            pl.BlockSpec((BLK, D), lambda i: (i, 0)),
            pl.BlockSpec((D, 64), full),
            pl.BlockSpec((D, 64), full),
            pl.BlockSpec((1, 64), full),
            pl.BlockSpec((64, 16), full),
            pl.BlockSpec((1, 16), full),
            pl.BlockSpec((1, 16), full),
            pl.BlockSpec((1, 1), full),
        ],
        out_specs=pl.BlockSpec((1, 1, BLK), lambda i: (i, 0, 0)),
        out_shape=jax.ShapeDtypeStruct((n // BLK, 1, BLK), jnp.float32),
        compiler_params=pltpu.CompilerParams(
            dimension_semantics=("parallel",)),
    )(eu, ei, w1a, w1b, b1, w2, b2, wp, bp)


def kernel(user, item, embed_user, embed_item, W1, b1, W2, b2, Wp, bp):
    user = user.astype(jnp.int32)
    item = item.astype(jnp.int32)
    w1a = W1[:D]
    w1b = W1[D:]
    b1r = b1.reshape(1, 64)
    b2r = b2.reshape(1, 16)
    wpr = Wp.reshape(1, 16)
    bpr = bp.reshape(1, 1)
    h = B // 2
    preds = []
    for lo in (0, h):
        eu, ei = _gather(lax.dynamic_slice_in_dim(user, lo, h),
                         lax.dynamic_slice_in_dim(item, lo, h),
                         embed_user, embed_item)
        preds.append(_mlp(eu, ei, w1a, w1b, b1r, W2, b2r, wpr, bpr))
    return jnp.concatenate(preds).reshape(-1)
